# baseline (speedup 1.0000x reference)
import functools

import jax
import jax.numpy as jnp
from jax import lax
from jax.experimental import pallas as pl
from jax.experimental.pallas import tpu as pltpu
from jax.experimental.pallas import tpu_sc as plsc


def _build(NW, NG, G, L, E, V):
  mesh = plsc.VectorSubcoreMesh(core_axis_name="c", subcore_axis_name="s")
  NC = plsc.get_sparse_core_info().num_cores

  @functools.partial(
      pl.kernel,
      out_type=jax.ShapeDtypeStruct((NW * NG * G, E), jnp.float32),
      mesh=mesh,
      compiler_params=pltpu.CompilerParams(use_tc_tiling_on_sc=False),
      scratch_types=[
          pltpu.VMEM((NG, G), jnp.int32),
          pltpu.VMEM((G, E), jnp.float32),
          pltpu.VMEM_SHARED((2 * L, E), jnp.float32),
          pltpu.SemaphoreType.DMA,
      ],
  )
  def k(idx_hbm, tok_hbm, pos_hbm, out_hbm, idx_v, rows, pos_sh, sem):
    c = lax.axis_index("c")
    s = lax.axis_index("s")
    wid = s * NC + c

    @pl.when(s == 0)
    def _():
      pltpu.sync_copy(pos_hbm.at[pl.ds(0, L)], pos_sh.at[pl.ds(0, L)])
      pltpu.sync_copy(pos_hbm.at[pl.ds(0, L)], pos_sh.at[pl.ds(L, L)])

    plsc.subcore_barrier()

    pltpu.sync_copy(idx_hbm.at[wid], idx_v)

    base = wid * (NG * G)

    def group(j, carry):
      o = lax.rem(j * G, L)
      pltpu.sync_copy(pos_sh.at[pl.ds(o, G)], rows)
      pltpu.async_copy(tok_hbm.at[idx_v.at[j]], rows, sem, add=True).wait()
      pltpu.sync_copy(rows, out_hbm.at[pl.ds(base + j * G, G)])
      return carry

    lax.fori_loop(0, NG, group, 0)

  return k


def kernel(in_idx, tok_table, pos_table):
  B, L = in_idx.shape
  V, E = tok_table.shape
  info = plsc.get_sparse_core_info()
  NW = info.num_cores * info.num_subcores
  G = 128
  TOK = B * L
  per_w = TOK // NW
  NG = per_w // G
  assert per_w * NW == TOK and NG * G == per_w

  idx3 = in_idx.reshape(NW, NG, G).astype(jnp.int32)
  out = _build(NW, NG, G, L, E, V)(idx3, tok_table, pos_table)
  return out.reshape(B, L, E)


# speedup vs baseline: 1.1140x; 1.1140x over previous
import functools

import jax
import jax.numpy as jnp
from jax import lax
from jax.experimental import pallas as pl
from jax.experimental.pallas import tpu as pltpu
from jax.experimental.pallas import tpu_sc as plsc

NBUF = 4


def _build(NW, NG, G, L, E):
  mesh = plsc.VectorSubcoreMesh(core_axis_name="c", subcore_axis_name="s")
  NC = plsc.get_sparse_core_info().num_cores

  @functools.partial(
      pl.kernel,
      out_type=jax.ShapeDtypeStruct((NW * NG * G, E), jnp.float32),
      mesh=mesh,
      compiler_params=pltpu.CompilerParams(use_tc_tiling_on_sc=False),
      scratch_types=[
          pltpu.VMEM((NG, G), jnp.int32),
          pltpu.VMEM((NBUF, G, E), jnp.float32),
          pltpu.VMEM_SHARED((2 * L, E), jnp.float32),
          pltpu.SemaphoreType.DMA((NBUF,)),
          pltpu.SemaphoreType.DMA((NBUF,)),
          pltpu.SemaphoreType.DMA((NBUF,)),
      ],
  )
  def k(idx_hbm, tok_hbm, pos_hbm, out_hbm, idx_v, rows, pos_sh,
        sem_a, sem_b, sem_c):
    c = lax.axis_index("c")
    s = lax.axis_index("s")
    wid = s * NC + c

    @pl.when(s == 0)
    def _():
      pltpu.sync_copy(pos_hbm.at[pl.ds(0, L)], pos_sh.at[pl.ds(0, L)])
      pltpu.sync_copy(pos_hbm.at[pl.ds(0, L)], pos_sh.at[pl.ds(L, L)])

    plsc.subcore_barrier()

    pltpu.sync_copy(idx_hbm.at[wid], idx_v)

    base = wid * (NG * G)

    def step(j, carry):
      @pl.when(j < NG)
      def _():
        r = lax.rem(j, NBUF)
        @pl.when(j >= NBUF)
        def _():
          pltpu.make_async_copy(rows.at[r], out_hbm.at[pl.ds(0, G)],
                                sem_c.at[r]).wait()
        o = lax.rem(j * G, L)
        pltpu.async_copy(pos_sh.at[pl.ds(o, G)], rows.at[r], sem_a.at[r])

      jb = j - 1
      @pl.when((jb >= 0) & (jb < NG))
      def _():
        r = lax.rem(jb, NBUF)
        pltpu.make_async_copy(pos_sh.at[pl.ds(0, G)], rows.at[r],
                              sem_a.at[r]).wait()
        pltpu.async_copy(tok_hbm.at[idx_v.at[jb]], rows.at[r],
                         sem_b.at[r], add=True)

      jc = j - 2
      @pl.when((jc >= 0) & (jc < NG))
      def _():
        r = lax.rem(jc, NBUF)
        pltpu.make_async_copy(tok_hbm.at[idx_v.at[0]], rows.at[r],
                              sem_b.at[r]).wait()
        pltpu.async_copy(rows.at[r], out_hbm.at[pl.ds(base + jc * G, G)],
                         sem_c.at[r])

      return carry

    lax.fori_loop(0, NG + 2, step, 0)

    for b in range(NBUF):
      pltpu.make_async_copy(rows.at[b], out_hbm.at[pl.ds(0, G)],
                            sem_c.at[b]).wait()

  return k


def kernel(in_idx, tok_table, pos_table):
  B, L = in_idx.shape
  V, E = tok_table.shape
  info = plsc.get_sparse_core_info()
  NW = info.num_cores * info.num_subcores
  G = 128
  TOK = B * L
  per_w = TOK // NW
  NG = per_w // G
  assert per_w * NW == TOK and NG * G == per_w

  idx3 = in_idx.reshape(NW, NG, G).astype(jnp.int32)
  out = _build(NW, NG, G, L, E)(idx3, tok_table, pos_table)
  return out.reshape(B, L, E)


# speedup vs baseline: 1.1502x; 1.0325x over previous
import functools

import jax
import jax.numpy as jnp
from jax import lax
from jax.experimental import pallas as pl
from jax.experimental.pallas import tpu as pltpu
from jax.experimental.pallas import tpu_sc as plsc

NBUF = 4


def _build(NW, NG, G, L, E):
  mesh = plsc.VectorSubcoreMesh(core_axis_name="c", subcore_axis_name="s")
  NC = plsc.get_sparse_core_info().num_cores

  @functools.partial(
      pl.kernel,
      out_type=jax.ShapeDtypeStruct((NW * NG * G, E), jnp.float32),
      mesh=mesh,
      compiler_params=pltpu.CompilerParams(use_tc_tiling_on_sc=False),
      scratch_types=[
          pltpu.VMEM((NG, G), jnp.int32),
          pltpu.VMEM((NBUF, G, E), jnp.float32),
          pltpu.SemaphoreType.DMA((NBUF,)),
          pltpu.SemaphoreType.DMA((NBUF,)),
          pltpu.SemaphoreType.DMA((NBUF,)),
      ],
  )
  def k(idx_hbm, tok_hbm, pos2_hbm, out_hbm, idx_v, rows,
        sem_a, sem_b, sem_c):
    c = lax.axis_index("c")
    s = lax.axis_index("s")
    wid = s * NC + c

    pltpu.sync_copy(idx_hbm.at[wid], idx_v)

    base = wid * (NG * G)

    def step(j, carry):
      @pl.when(j < NG)
      def _():
        r = lax.rem(j, NBUF)
        @pl.when(j >= NBUF)
        def _():
          pltpu.make_async_copy(rows.at[r], out_hbm.at[pl.ds(0, G)],
                                sem_c.at[r]).wait()
        o = lax.rem(j * G, L)
        pltpu.async_copy(pos2_hbm.at[pl.ds(o, G)], rows.at[r], sem_a.at[r])

      jb = j - 1
      @pl.when((jb >= 0) & (jb < NG))
      def _():
        r = lax.rem(jb, NBUF)
        pltpu.make_async_copy(pos2_hbm.at[pl.ds(0, G)], rows.at[r],
                              sem_a.at[r]).wait()
        pltpu.async_copy(tok_hbm.at[idx_v.at[jb]], rows.at[r],
                         sem_b.at[r], add=True)

      jc = j - 2
      @pl.when((jc >= 0) & (jc < NG))
      def _():
        r = lax.rem(jc, NBUF)
        pltpu.make_async_copy(tok_hbm.at[idx_v.at[0]], rows.at[r],
                              sem_b.at[r]).wait()
        pltpu.async_copy(rows.at[r], out_hbm.at[pl.ds(base + jc * G, G)],
                         sem_c.at[r])

      return carry

    lax.fori_loop(0, NG + 2, step, 0)

    for b in range(NBUF):
      pltpu.make_async_copy(rows.at[b], out_hbm.at[pl.ds(0, G)],
                            sem_c.at[b]).wait()

  return k


def kernel(in_idx, tok_table, pos_table):
  B, L = in_idx.shape
  V, E = tok_table.shape
  info = plsc.get_sparse_core_info()
  NW = info.num_cores * info.num_subcores
  G = 128
  TOK = B * L
  per_w = TOK // NW
  NG = per_w // G
  assert per_w * NW == TOK and NG * G == per_w

  idx3 = in_idx.reshape(NW, NG, G).astype(jnp.int32)
  pos2 = jnp.concatenate([pos_table[:L], pos_table[:L]], axis=0)
  out = _build(NW, NG, G, L, E)(idx3, tok_table, pos2)
  return out.reshape(B, L, E)


# speedup vs baseline: 1.4363x; 1.2487x over previous
import functools

import jax
import jax.numpy as jnp
from jax import lax
from jax.experimental import pallas as pl
from jax.experimental.pallas import tpu as pltpu
from jax.experimental.pallas import tpu_sc as plsc

NBUF = 2
LANES = 16


def _build(NW, L, B, E, V):
  mesh = plsc.VectorSubcoreMesh(core_axis_name="c", subcore_axis_name="s")
  NC = plsc.get_sparse_core_info().num_cores
  G = B // NW
  EV = E // LANES

  @functools.partial(
      pl.kernel,
      out_type=jax.ShapeDtypeStruct((L, E, B), jnp.float32),
      mesh=mesh,
      compiler_params=pltpu.CompilerParams(use_tc_tiling_on_sc=True,
                                           needs_layout_passes=False),
      scratch_types=[
          pltpu.VMEM((L, G), jnp.int32),
          pltpu.VMEM((NBUF, G), jnp.int32),
          pltpu.VMEM((NBUF, G, 128), jnp.float32),
          pltpu.VMEM((NBUF, E, G), jnp.float32),
          pltpu.VMEM((NBUF, 8, 128), jnp.float32),
          pltpu.SemaphoreType.DMA((NBUF,)),
          pltpu.SemaphoreType.DMA((NBUF,)),
          pltpu.SemaphoreType.DMA((NBUF,)),
      ],
  )
  def k(idx_hbm, tok2_hbm, poss_hbm, out_hbm, idx_v, idx2, rows, chunk,
        posb, sem_p, sem_b, sem_c):
    c = lax.axis_index("c")
    s = lax.axis_index("s")
    wid = s * NC + c
    b0 = wid * G

    pltpu.sync_copy(idx_hbm.at[:, pl.ds(b0, G)], idx_v)

    def step(j, carry):
      @pl.when(j < L)
      def _():
        r = lax.rem(j, NBUF)
        @pl.when(j >= NBUF)
        def _():
          pltpu.make_async_copy(chunk.at[r], out_hbm.at[0, :, pl.ds(0, G)],
                                sem_c.at[r]).wait()
        pltpu.async_copy(poss_hbm.at[j], posb.at[r], sem_p.at[r])
        for k8 in range(G // LANES):
          d = pl.ds(k8 * LANES, LANES)
          idx2[r, d] = lax.shift_right_logical(idx_v[j, d], 1)
        pltpu.async_copy(tok2_hbm.at[idx2.at[r]], rows.at[r], sem_b.at[r])

      jb = j - 1
      @pl.when((jb >= 0) & (jb < L))
      def _():
        r = lax.rem(jb, NBUF)
        pltpu.make_async_copy(poss_hbm.at[0], posb.at[r], sem_p.at[r]).wait()
        pltpu.make_async_copy(tok2_hbm.at[idx2.at[r]], rows.at[r],
                              sem_b.at[r]).wait()
        half = [
            (idx_v[jb, pl.ds(k8 * LANES, LANES)] & 1) * E
            for k8 in range(G // LANES)
        ]
        lane = lax.iota(jnp.int32, LANES)

        @plsc.parallel_loop(0, E, 1, unroll=8)
        def _(e):
          e8 = lax.div(e, 8)
          e1 = lax.rem(e, 8)
          pvec = posb[r, e8, pl.ds(e1 * LANES, LANES)]
          for k8 in range(G // LANES):
            col = half[k8] + e
            row = lane + (k8 * LANES)
            val = plsc.load_gather(rows.at[r], [row, col]) + pvec
            chunk[r, e, pl.ds(k8 * LANES, LANES)] = val

        pltpu.async_copy(chunk.at[r], out_hbm.at[jb, :, pl.ds(b0, G)],
                         sem_c.at[r])

      return carry

    lax.fori_loop(0, L + 1, step, 0)

    for rb in range(NBUF):
      pltpu.make_async_copy(chunk.at[rb], out_hbm.at[0, :, pl.ds(0, G)],
                            sem_c.at[rb]).wait()

  return k


def kernel(in_idx, tok_table, pos_table):
  B, L = in_idx.shape
  V, E = tok_table.shape
  info = plsc.get_sparse_core_info()
  NW = info.num_cores * info.num_subcores
  assert B % (NW * 128) == 0 or B == NW * 128

  idxT = in_idx.T.astype(jnp.int32)
  tok2 = tok_table.reshape(V // 2, 2 * E)
  poss = jnp.repeat(pos_table[:L].reshape(L, 8, E // 8), LANES, axis=2)
  out5 = _build(NW, L, B, E, V)(idxT, tok2, poss)
  return out5.transpose(2, 0, 1)
